# baseline (speedup 1.0000x reference)
import functools

import jax
import jax.numpy as jnp
from jax import lax
from jax.experimental import pallas as pl
from jax.experimental.pallas import tpu as pltpu
from jax.experimental.pallas import tpu_sc as plsc

_TOTAL_BITS = 4096
_NEURONS = 8192
_NBITS = 12
_BATCH = 1024
_LANES = 16
_WORDS = _BATCH // 8
_GROUPS = _WORDS // _LANES
_CHUNK = 4


@functools.lru_cache(maxsize=None)
def _build_sc_kernel():
    info = plsc.get_sparse_core_info()
    nc, ns = info.num_cores, info.num_subcores
    nw = nc * ns
    npt = _NEURONS // nw
    nchunks = npt // _CHUNK
    crow = _CHUNK * _NBITS
    mesh = plsc.VectorSubcoreMesh(core_axis_name="c", subcore_axis_name="s")

    @functools.partial(
        pl.kernel,
        mesh=mesh,
        compiler_params=pltpu.CompilerParams(needs_layout_passes=False),
        out_type=jax.ShapeDtypeStruct((_NEURONS, _BATCH), jnp.float32),
        scratch_types=[
            pltpu.VMEM_SHARED((_TOTAL_BITS, _WORDS), jnp.int32),
            pltpu.VMEM((npt * _NBITS,), jnp.int32),
            pltpu.VMEM((crow, _WORDS), jnp.int32),
            pltpu.VMEM((crow, _WORDS), jnp.int32),
            pltpu.VMEM((_CHUNK, 2 ** _NBITS), jnp.float32),
            pltpu.VMEM((_CHUNK, 2 ** _NBITS), jnp.float32),
            pltpu.VMEM((_CHUNK, _BATCH), jnp.float32),
            pltpu.VMEM((_CHUNK, _BATCH), jnp.float32),
            pltpu.SemaphoreType.DMA,
            pltpu.SemaphoreType.DMA,
            pltpu.SemaphoreType.DMA,
            pltpu.SemaphoreType.DMA,
            pltpu.SemaphoreType.DMA,
            pltpu.SemaphoreType.DMA,
        ],
    )
    def ram_kernel(bits_hbm, conn_hbm, mem_hbm, out_hbm,
                   bits_sh, conn_v, rows_a, rows_b, mem_a, mem_b, out_a, out_b,
                   sin_a, sin_b, smem_a, smem_b, sout_a, sout_b):
        sid = lax.axis_index("s")
        wid = sid * nc + lax.axis_index("c")
        n0 = wid * npt
        shard = _TOTAL_BITS // ns
        pltpu.sync_copy(bits_hbm.at[pl.ds(sid * shard, shard)],
                        bits_sh.at[pl.ds(sid * shard, shard)])
        pltpu.sync_copy(conn_hbm.at[pl.ds(n0 * _NBITS, npt * _NBITS)], conn_v)
        plsc.subcore_barrier()

        rows = (rows_a, rows_b)
        mem = (mem_a, mem_b)
        out = (out_a, out_b)
        sin = (sin_a, sin_b)
        smem = (smem_a, smem_b)
        sout = (sout_a, sout_b)

        def issue_in(ci, b):
            pltpu.make_async_copy(
                bits_sh.at[conn_v.at[pl.ds(ci * crow, crow)]], rows[b], sin[b]
            ).start()
            pltpu.make_async_copy(
                mem_hbm.at[pl.ds(n0 + ci * _CHUNK, _CHUNK)], mem[b], smem[b]
            ).start()

        def wait_in(b):
            pltpu.make_async_copy(
                bits_sh.at[pl.ds(0, crow)], rows[b], sin[b]).wait()
            pltpu.make_async_copy(
                mem_hbm.at[pl.ds(0, _CHUNK)], mem[b], smem[b]).wait()

        def issue_out(ci, b):
            pltpu.make_async_copy(
                out[b], out_hbm.at[pl.ds(n0 + ci * _CHUNK, _CHUNK)], sout[b]
            ).start()

        def wait_out(b):
            pltpu.make_async_copy(
                out[b], out_hbm.at[pl.ds(0, _CHUNK)], sout[b]).wait()

        def compute(b):
            for q in range(_CHUNK):
                qvec = jnp.full((_LANES,), q, jnp.int32)
                for t in range(_GROUPS):
                    n0a = jnp.zeros((_LANES,), jnp.int32)
                    n1a = jnp.zeros((_LANES,), jnp.int32)
                    n2a = jnp.zeros((_LANES,), jnp.int32)
                    for k in range(_NBITS):
                        w = rows[b][q * _NBITS + k, pl.ds(t * _LANES, _LANES)]
                        if k < 4:
                            n0a = n0a + (w << k)
                        elif k < 8:
                            n1a = n1a + (w << (k - 4))
                        else:
                            n2a = n2a + (w << (k - 8))
                    for j in range(8):
                        addr = (((n0a >> (4 * j)) & 0xF)
                                | (((n1a >> (4 * j)) & 0xF) << 4)
                                | (((n2a >> (4 * j)) & 0xF) << 8))
                        vals = plsc.load_gather(mem[b], [qvec, addr])
                        out[b][q, pl.ds(t * 128 + j * _LANES, _LANES)] = vals

        issue_in(0, 0)

        def body(h, carry):
            c0 = h * 2
            issue_in(c0 + 1, 1)
            wait_in(0)

            @pl.when(h > 0)
            def _():
                wait_out(0)

            compute(0)
            issue_out(c0, 0)

            @pl.when(h < nchunks // 2 - 1)
            def _():
                issue_in(c0 + 2, 0)

            wait_in(1)

            @pl.when(h > 0)
            def _():
                wait_out(1)

            compute(1)
            issue_out(c0 + 1, 1)
            return carry

        lax.fori_loop(0, nchunks // 2, body, 0)
        wait_out(0)
        wait_out(1)

    return ram_kernel


def kernel(input_bits, connections, memory):
    bits8_t = input_bits.astype(jnp.int8).T
    pair = bits8_t.reshape(_TOTAL_BITS, _BATCH // 2, 2)
    nib = pair[:, :, 0] | (pair[:, :, 1] << 4)
    bits_packed = jax.lax.bitcast_convert_type(
        nib.reshape(_TOTAL_BITS, _WORDS, 4), jnp.int32)
    conn_flat = connections.reshape(-1)
    vals = _build_sc_kernel()(bits_packed, conn_flat, memory)
    vals = vals.reshape(_NEURONS, _GROUPS, 8, _LANES).swapaxes(2, 3)
    return vals.reshape(_NEURONS, _BATCH).T > 0.5


# speedup vs baseline: 1.1922x; 1.1922x over previous
import functools

import jax
import jax.numpy as jnp
from jax import lax
from jax.experimental import pallas as pl
from jax.experimental.pallas import tpu as pltpu
from jax.experimental.pallas import tpu_sc as plsc

_TOTAL_BITS = 4096
_NEURONS = 8192
_NBITS = 12
_BATCH = 1024
_LANES = 16
_WORDS = _BATCH // 8
_GROUPS = _WORDS // _LANES
_CHUNK = 4


@functools.lru_cache(maxsize=None)
def _build_sc_kernel():
    info = plsc.get_sparse_core_info()
    nc, ns = info.num_cores, info.num_subcores
    nw = nc * ns
    npt = _NEURONS // nw
    nchunks = npt // _CHUNK
    crow = _CHUNK * _NBITS
    mesh = plsc.VectorSubcoreMesh(core_axis_name="c", subcore_axis_name="s")

    @functools.partial(
        pl.kernel,
        mesh=mesh,
        compiler_params=pltpu.CompilerParams(needs_layout_passes=False),
        out_type=jax.ShapeDtypeStruct((_NEURONS, _BATCH), jnp.float32),
        scratch_types=[
            pltpu.VMEM_SHARED((_TOTAL_BITS, _WORDS), jnp.int32),
            pltpu.VMEM((npt * _NBITS,), jnp.int32),
            pltpu.VMEM((crow, _WORDS), jnp.int32),
            pltpu.VMEM((crow, _WORDS), jnp.int32),
            pltpu.VMEM((_CHUNK, 2 ** _NBITS), jnp.float32),
            pltpu.VMEM((_CHUNK, 2 ** _NBITS), jnp.float32),
            pltpu.VMEM((_CHUNK, _BATCH), jnp.float32),
            pltpu.VMEM((_CHUNK, _BATCH), jnp.float32),
            pltpu.SemaphoreType.DMA,
            pltpu.SemaphoreType.DMA,
            pltpu.SemaphoreType.DMA,
            pltpu.SemaphoreType.DMA,
            pltpu.SemaphoreType.DMA,
            pltpu.SemaphoreType.DMA,
        ],
    )
    def ram_kernel(bits_hbm, conn_hbm, mem_hbm, out_hbm,
                   bits_sh, conn_v, rows_a, rows_b, mem_a, mem_b, out_a, out_b,
                   sin_a, sin_b, smem_a, smem_b, sout_a, sout_b):
        sid = lax.axis_index("s")
        wid = sid * nc + lax.axis_index("c")
        n0 = wid * npt
        shard = _TOTAL_BITS // ns
        pltpu.sync_copy(bits_hbm.at[pl.ds(sid * shard, shard)],
                        bits_sh.at[pl.ds(sid * shard, shard)])
        pltpu.sync_copy(conn_hbm.at[pl.ds(n0 * _NBITS, npt * _NBITS)], conn_v)
        plsc.subcore_barrier()

        rows = (rows_a, rows_b)
        mem = (mem_a, mem_b)
        out = (out_a, out_b)
        sin = (sin_a, sin_b)
        smem = (smem_a, smem_b)
        sout = (sout_a, sout_b)

        def issue_in(ci, b):
            pltpu.make_async_copy(
                bits_sh.at[conn_v.at[pl.ds(ci * crow, crow)]], rows[b], sin[b]
            ).start()
            pltpu.make_async_copy(
                mem_hbm.at[pl.ds(n0 + ci * _CHUNK, _CHUNK)], mem[b], smem[b]
            ).start()

        def wait_in(b):
            pltpu.make_async_copy(
                bits_sh.at[pl.ds(0, crow)], rows[b], sin[b]).wait()
            pltpu.make_async_copy(
                mem_hbm.at[pl.ds(0, _CHUNK)], mem[b], smem[b]).wait()

        def issue_out(ci, b):
            pltpu.make_async_copy(
                out[b], out_hbm.at[pl.ds(n0 + ci * _CHUNK, _CHUNK)], sout[b]
            ).start()

        def wait_out(b):
            pltpu.make_async_copy(
                out[b], out_hbm.at[pl.ds(0, _CHUNK)], sout[b]).wait()

        def compute(b):
            for q in range(_CHUNK):
                qvec = jnp.full((_LANES,), q, jnp.int32)

                @plsc.parallel_loop(0, _GROUPS, 1, unroll=2)
                def _group(t):
                    n0a = n1a = n2a = None
                    for k in range(_NBITS):
                        w = rows[b][q * _NBITS + k, pl.ds(t * _LANES, _LANES)]
                        if k < 4:
                            n0a = (w << k) if n0a is None else n0a + (w << k)
                        elif k < 8:
                            n1a = (w << (k - 4)) if n1a is None else n1a + (w << (k - 4))
                        else:
                            n2a = (w << (k - 8)) if n2a is None else n2a + (w << (k - 8))
                    for j in range(8):
                        p0 = (n0a >> (4 * j)) & 0xF if j else n0a & 0xF
                        if j == 0:
                            p1 = (n1a << 4) & 0xF0
                            p2 = (n2a << 8) & 0xF00
                        elif j == 1:
                            p1 = n1a & 0xF0
                            p2 = (n2a << 4) & 0xF00
                        else:
                            p1 = (n1a >> (4 * j - 4)) & 0xF0
                            p2 = (n2a >> (4 * j - 8)) & 0xF00
                        addr = p0 | p1 | p2
                        vals = plsc.load_gather(mem[b], [qvec, addr])
                        out[b][q, pl.ds(t * 128 + j * _LANES, _LANES)] = vals

        issue_in(0, 0)

        def body(h, carry):
            c0 = h * 2
            issue_in(c0 + 1, 1)
            wait_in(0)

            @pl.when(h > 0)
            def _():
                wait_out(0)

            compute(0)
            issue_out(c0, 0)

            @pl.when(h < nchunks // 2 - 1)
            def _():
                issue_in(c0 + 2, 0)

            wait_in(1)

            @pl.when(h > 0)
            def _():
                wait_out(1)

            compute(1)
            issue_out(c0 + 1, 1)
            return carry

        lax.fori_loop(0, nchunks // 2, body, 0)
        wait_out(0)
        wait_out(1)

    return ram_kernel


def kernel(input_bits, connections, memory):
    bits8_t = input_bits.astype(jnp.int8).T
    pair = bits8_t.reshape(_TOTAL_BITS, _BATCH // 2, 2)
    nib = pair[:, :, 0] | (pair[:, :, 1] << 4)
    bits_packed = jax.lax.bitcast_convert_type(
        nib.reshape(_TOTAL_BITS, _WORDS, 4), jnp.int32)
    conn_flat = connections.reshape(-1)
    vals = _build_sc_kernel()(bits_packed, conn_flat, memory)
    vals = vals.reshape(_NEURONS, _GROUPS, 8, _LANES).swapaxes(2, 3)
    return vals.reshape(_NEURONS, _BATCH).T > 0.5


# speedup vs baseline: 1.3708x; 1.1498x over previous
import functools

import jax
import jax.numpy as jnp
from jax import lax
from jax.experimental import pallas as pl
from jax.experimental.pallas import tpu as pltpu
from jax.experimental.pallas import tpu_sc as plsc

_TOTAL_BITS = 4096
_NEURONS = 8192
_NBITS = 12
_BATCH = 1024
_LANES = 16
_WORDS = _BATCH // 8
_GROUPS = _WORDS // _LANES
_CHUNK = 4


@functools.lru_cache(maxsize=None)
def _build_sc_kernel():
    info = plsc.get_sparse_core_info()
    nc, ns = info.num_cores, info.num_subcores
    nw = nc * ns
    npt = _NEURONS // nw
    nchunks = npt // _CHUNK
    crow = _CHUNK * _NBITS
    mesh = plsc.VectorSubcoreMesh(core_axis_name="c", subcore_axis_name="s")

    @functools.partial(
        pl.kernel,
        mesh=mesh,
        compiler_params=pltpu.CompilerParams(needs_layout_passes=False),
        out_type=jax.ShapeDtypeStruct((_NEURONS, _BATCH), jnp.float32),
        scratch_types=[
            pltpu.VMEM_SHARED((_TOTAL_BITS, _WORDS), jnp.int32),
            pltpu.VMEM((npt * _NBITS,), jnp.int32),
            pltpu.VMEM((crow, _WORDS), jnp.int32),
            pltpu.VMEM((crow, _WORDS), jnp.int32),
            pltpu.VMEM((_CHUNK, 2 ** _NBITS), jnp.float32),
            pltpu.VMEM((_CHUNK, 2 ** _NBITS), jnp.float32),
            pltpu.VMEM((_CHUNK, _BATCH), jnp.float32),
            pltpu.VMEM((_CHUNK, _BATCH), jnp.float32),
            pltpu.SemaphoreType.DMA,
            pltpu.SemaphoreType.DMA,
            pltpu.SemaphoreType.DMA,
            pltpu.SemaphoreType.DMA,
            pltpu.SemaphoreType.DMA,
            pltpu.SemaphoreType.DMA,
        ],
    )
    def ram_kernel(bits_hbm, conn_hbm, mem_hbm, out_hbm,
                   bits_sh, conn_v, rows_a, rows_b, mem_a, mem_b, out_a, out_b,
                   sin_a, sin_b, smem_a, smem_b, sout_a, sout_b):
        sid = lax.axis_index("s")
        wid = sid * nc + lax.axis_index("c")
        n0 = wid * npt
        shard = _TOTAL_BITS // ns
        pltpu.sync_copy(bits_hbm.at[pl.ds(sid * shard, shard)],
                        bits_sh.at[pl.ds(sid * shard, shard)])
        pltpu.sync_copy(conn_hbm.at[pl.ds(n0 * _NBITS, npt * _NBITS)], conn_v)
        plsc.subcore_barrier()

        rows = (rows_a, rows_b)
        mem = (mem_a, mem_b)
        out = (out_a, out_b)
        sin = (sin_a, sin_b)
        smem = (smem_a, smem_b)
        sout = (sout_a, sout_b)

        def issue_in(ci, b):
            pltpu.make_async_copy(
                bits_sh.at[conn_v.at[pl.ds(ci * crow, crow)]], rows[b], sin[b]
            ).start()
            pltpu.make_async_copy(
                mem_hbm.at[pl.ds(n0 + ci * _CHUNK, _CHUNK)], mem[b], smem[b]
            ).start()

        def wait_in(b):
            pltpu.make_async_copy(
                bits_sh.at[pl.ds(0, crow)], rows[b], sin[b]).wait()
            pltpu.make_async_copy(
                mem_hbm.at[pl.ds(0, _CHUNK)], mem[b], smem[b]).wait()

        def issue_out(ci, b):
            pltpu.make_async_copy(
                out[b], out_hbm.at[pl.ds(n0 + ci * _CHUNK, _CHUNK)], sout[b]
            ).start()

        def wait_out(b):
            pltpu.make_async_copy(
                out[b], out_hbm.at[pl.ds(0, _CHUNK)], sout[b]).wait()

        def compute(b):
            @plsc.parallel_loop(0, _CHUNK * _GROUPS, 1, unroll=2)
            def _group(i):
                q = i // _GROUPS
                t = i % _GROUPS
                qvec = jnp.broadcast_to(q, (_LANES,)).astype(jnp.int32)
                if True:
                    n0a = n1a = n2a = None
                    for k in range(_NBITS):
                        w = rows[b][q * _NBITS + k, pl.ds(t * _LANES, _LANES)]
                        if k < 4:
                            n0a = (w << k) if n0a is None else n0a + (w << k)
                        elif k < 8:
                            n1a = (w << (k - 4)) if n1a is None else n1a + (w << (k - 4))
                        else:
                            n2a = (w << (k - 8)) if n2a is None else n2a + (w << (k - 8))
                    for j in range(8):
                        p0 = (n0a >> (4 * j)) & 0xF if j else n0a & 0xF
                        if j == 0:
                            p1 = (n1a << 4) & 0xF0
                            p2 = (n2a << 8) & 0xF00
                        elif j == 1:
                            p1 = n1a & 0xF0
                            p2 = (n2a << 4) & 0xF00
                        else:
                            p1 = (n1a >> (4 * j - 4)) & 0xF0
                            p2 = (n2a >> (4 * j - 8)) & 0xF00
                        addr = p0 | p1 | p2
                        vals = plsc.load_gather(mem[b], [qvec, addr])
                        out[b][q, pl.ds(t * 128 + j * _LANES, _LANES)] = vals

        issue_in(0, 0)

        def body(h, carry):
            c0 = h * 2
            issue_in(c0 + 1, 1)
            wait_in(0)

            @pl.when(h > 0)
            def _():
                wait_out(0)

            compute(0)
            issue_out(c0, 0)

            @pl.when(h < nchunks // 2 - 1)
            def _():
                issue_in(c0 + 2, 0)

            wait_in(1)

            @pl.when(h > 0)
            def _():
                wait_out(1)

            compute(1)
            issue_out(c0 + 1, 1)
            return carry

        lax.fori_loop(0, nchunks // 2, body, 0)
        wait_out(0)
        wait_out(1)

    return ram_kernel


def kernel(input_bits, connections, memory):
    bits8_t = input_bits.astype(jnp.int8).T
    pair = bits8_t.reshape(_TOTAL_BITS, _BATCH // 2, 2)
    nib = pair[:, :, 0] | (pair[:, :, 1] << 4)
    bits_packed = jax.lax.bitcast_convert_type(
        nib.reshape(_TOTAL_BITS, _WORDS, 4), jnp.int32)
    conn_flat = connections.reshape(-1)
    vals = _build_sc_kernel()(bits_packed, conn_flat, memory)
    vals = vals.reshape(_NEURONS, _GROUPS, 8, _LANES).swapaxes(2, 3)
    return vals.reshape(_NEURONS, _BATCH).T > 0.5


# speedup vs baseline: 1.5975x; 1.1653x over previous
import functools

import jax
import jax.numpy as jnp
from jax import lax
from jax.experimental import pallas as pl
from jax.experimental.pallas import tpu as pltpu
from jax.experimental.pallas import tpu_sc as plsc

_TOTAL_BITS = 4096
_NEURONS = 8192
_NBITS = 12
_BATCH = 1024
_LANES = 16
_WORDS = _BATCH // 8
_GROUPS = _WORDS // _LANES
_CHUNK = 4


@functools.lru_cache(maxsize=None)
def _build_sc_kernel():
    info = plsc.get_sparse_core_info()
    nc, ns = info.num_cores, info.num_subcores
    nw = nc * ns
    npt = _NEURONS // nw
    nchunks = npt // _CHUNK
    crow = _CHUNK * _NBITS
    mesh = plsc.VectorSubcoreMesh(core_axis_name="c", subcore_axis_name="s")

    @functools.partial(
        pl.kernel,
        mesh=mesh,
        compiler_params=pltpu.CompilerParams(needs_layout_passes=False),
        out_type=jax.ShapeDtypeStruct((_NEURONS, _BATCH // 4), jnp.int32),
        scratch_types=[
            pltpu.VMEM_SHARED((_TOTAL_BITS, _WORDS), jnp.int32),
            pltpu.VMEM((npt * _NBITS,), jnp.int32),
            pltpu.VMEM((crow, _WORDS), jnp.int32),
            pltpu.VMEM((crow, _WORDS), jnp.int32),
            pltpu.VMEM((_CHUNK, 2 ** _NBITS), jnp.float32),
            pltpu.VMEM((_CHUNK, 2 ** _NBITS), jnp.float32),
            pltpu.VMEM((_CHUNK, _BATCH // 4), jnp.int32),
            pltpu.VMEM((_CHUNK, _BATCH // 4), jnp.int32),
            pltpu.SemaphoreType.DMA,
            pltpu.SemaphoreType.DMA,
            pltpu.SemaphoreType.DMA,
            pltpu.SemaphoreType.DMA,
            pltpu.SemaphoreType.DMA,
            pltpu.SemaphoreType.DMA,
        ],
    )
    def ram_kernel(bits_hbm, conn_hbm, mem_hbm, out_hbm,
                   bits_sh, conn_v, rows_a, rows_b, mem_a, mem_b, out_a, out_b,
                   sin_a, sin_b, smem_a, smem_b, sout_a, sout_b):
        sid = lax.axis_index("s")
        wid = sid * nc + lax.axis_index("c")
        n0 = wid * npt
        shard = _TOTAL_BITS // ns
        pltpu.sync_copy(bits_hbm.at[pl.ds(sid * shard, shard)],
                        bits_sh.at[pl.ds(sid * shard, shard)])
        pltpu.sync_copy(conn_hbm.at[pl.ds(n0 * _NBITS, npt * _NBITS)], conn_v)
        plsc.subcore_barrier()

        rows = (rows_a, rows_b)
        mem = (mem_a, mem_b)
        out = (out_a, out_b)
        sin = (sin_a, sin_b)
        smem = (smem_a, smem_b)
        sout = (sout_a, sout_b)

        def issue_in(ci, b):
            pltpu.make_async_copy(
                bits_sh.at[conn_v.at[pl.ds(ci * crow, crow)]], rows[b], sin[b]
            ).start()
            pltpu.make_async_copy(
                mem_hbm.at[pl.ds(n0 + ci * _CHUNK, _CHUNK)], mem[b], smem[b]
            ).start()

        def wait_in(b):
            pltpu.make_async_copy(
                bits_sh.at[pl.ds(0, crow)], rows[b], sin[b]).wait()
            pltpu.make_async_copy(
                mem_hbm.at[pl.ds(0, _CHUNK)], mem[b], smem[b]).wait()

        def issue_out(ci, b):
            pltpu.make_async_copy(
                out[b], out_hbm.at[pl.ds(n0 + ci * _CHUNK, _CHUNK)], sout[b]
            ).start()

        def wait_out(b):
            pltpu.make_async_copy(
                out[b], out_hbm.at[pl.ds(0, _CHUNK)], sout[b]).wait()

        def compute(b):
            @plsc.parallel_loop(0, _CHUNK * _GROUPS, 1, unroll=2)
            def _group(i):
                q = i // _GROUPS
                t = i % _GROUPS
                qvec = jnp.broadcast_to(q, (_LANES,)).astype(jnp.int32)
                if True:
                    n0a = n1a = n2a = None
                    for k in range(_NBITS):
                        w = rows[b][q * _NBITS + k, pl.ds(t * _LANES, _LANES)]
                        if k < 4:
                            n0a = (w << k) if n0a is None else n0a + (w << k)
                        elif k < 8:
                            n1a = (w << (k - 4)) if n1a is None else n1a + (w << (k - 4))
                        else:
                            n2a = (w << (k - 8)) if n2a is None else n2a + (w << (k - 8))
                    for h in range(2):
                        word = None
                        for s in range(4):
                            j = 4 * h + s
                            p0 = (n0a >> (4 * j)) & 0xF if j else n0a & 0xF
                            if j == 0:
                                p1 = (n1a << 4) & 0xF0
                                p2 = (n2a << 8) & 0xF00
                            elif j == 1:
                                p1 = n1a & 0xF0
                                p2 = (n2a << 4) & 0xF00
                            else:
                                p1 = (n1a >> (4 * j - 4)) & 0xF0
                                p2 = (n2a >> (4 * j - 8)) & 0xF00
                            addr = p0 | p1 | p2
                            vals = plsc.load_gather(mem[b], [qvec, addr])
                            bit = jnp.where(vals > 0.5, 1, 0).astype(jnp.int32)
                            word = bit if s == 0 else word | (bit << (8 * s))
                        out[b][q, pl.ds(t * 32 + h * _LANES, _LANES)] = word

        issue_in(0, 0)

        def body(h, carry):
            c0 = h * 2
            issue_in(c0 + 1, 1)
            wait_in(0)

            @pl.when(h > 0)
            def _():
                wait_out(0)

            compute(0)
            issue_out(c0, 0)

            @pl.when(h < nchunks // 2 - 1)
            def _():
                issue_in(c0 + 2, 0)

            wait_in(1)

            @pl.when(h > 0)
            def _():
                wait_out(1)

            compute(1)
            issue_out(c0 + 1, 1)
            return carry

        lax.fori_loop(0, nchunks // 2, body, 0)
        wait_out(0)
        wait_out(1)

    return ram_kernel


def kernel(input_bits, connections, memory):
    shifts = (jnp.arange(8, dtype=jnp.int32) * 4)[None, :, None]
    bits_packed = (
        (input_bits.reshape(_WORDS, 8, _TOTAL_BITS) << shifts)
        .sum(axis=1, dtype=jnp.int32).T)
    conn_flat = connections.reshape(-1)
    packed = _build_sc_kernel()(bits_packed, conn_flat, memory)
    by = jax.lax.bitcast_convert_type(packed, jnp.int8)
    by = by.reshape(_NEURONS, _GROUPS, 2, _LANES, 4).transpose(0, 1, 3, 2, 4)
    return by.reshape(_NEURONS, _BATCH).T.astype(jnp.bool_)


# speedup vs baseline: 1.6616x; 1.0402x over previous
import functools

import jax
import jax.numpy as jnp
from jax import lax
from jax.experimental import pallas as pl
from jax.experimental.pallas import tpu as pltpu
from jax.experimental.pallas import tpu_sc as plsc

_TOTAL_BITS = 4096
_NEURONS = 8192
_NBITS = 12
_BATCH = 1024
_LANES = 16
_WORDS = _BATCH // 8
_GROUPS = _WORDS // _LANES
_CHUNK = 4


@functools.lru_cache(maxsize=None)
def _build_sc_kernel():
    info = plsc.get_sparse_core_info()
    nc, ns = info.num_cores, info.num_subcores
    nw = nc * ns
    npt = _NEURONS // nw
    nchunks = npt // _CHUNK
    crow = _CHUNK * _NBITS
    mesh = plsc.VectorSubcoreMesh(core_axis_name="c", subcore_axis_name="s")

    @functools.partial(
        pl.kernel,
        mesh=mesh,
        compiler_params=pltpu.CompilerParams(needs_layout_passes=False),
        out_type=jax.ShapeDtypeStruct((_NEURONS, _BATCH // 4), jnp.int32),
        scratch_types=[
            pltpu.VMEM_SHARED((_TOTAL_BITS, _WORDS), jnp.int32),
            pltpu.VMEM((npt * _NBITS,), jnp.int32),
            pltpu.VMEM((crow, _WORDS), jnp.int32),
            pltpu.VMEM((crow, _WORDS), jnp.int32),
            pltpu.VMEM((_CHUNK, 2 ** _NBITS), jnp.float32),
            pltpu.VMEM((_CHUNK, 2 ** _NBITS), jnp.float32),
            pltpu.VMEM((_CHUNK, _BATCH // 4), jnp.int32),
            pltpu.VMEM((_CHUNK, _BATCH // 4), jnp.int32),
            pltpu.SemaphoreType.DMA,
            pltpu.SemaphoreType.DMA,
            pltpu.SemaphoreType.DMA,
            pltpu.SemaphoreType.DMA,
            pltpu.SemaphoreType.DMA,
            pltpu.SemaphoreType.DMA,
        ],
    )
    def ram_kernel(bits_hbm, conn_hbm, mem_hbm, out_hbm,
                   bits_sh, conn_v, rows_a, rows_b, mem_a, mem_b, out_a, out_b,
                   sin_a, sin_b, smem_a, smem_b, sout_a, sout_b):
        sid = lax.axis_index("s")
        wid = sid * nc + lax.axis_index("c")
        n0 = wid * npt
        shard = _TOTAL_BITS // ns
        pltpu.sync_copy(bits_hbm.at[pl.ds(sid * shard, shard)],
                        bits_sh.at[pl.ds(sid * shard, shard)])
        pltpu.sync_copy(conn_hbm.at[pl.ds(n0 * _NBITS, npt * _NBITS)], conn_v)
        plsc.subcore_barrier()

        rows = (rows_a, rows_b)
        mem = (mem_a, mem_b)
        out = (out_a, out_b)
        sin = (sin_a, sin_b)
        smem = (smem_a, smem_b)
        sout = (sout_a, sout_b)

        def issue_in(ci, b):
            pltpu.make_async_copy(
                bits_sh.at[conn_v.at[pl.ds(ci * crow, crow)]], rows[b], sin[b]
            ).start()
            pltpu.make_async_copy(
                mem_hbm.at[pl.ds(n0 + ci * _CHUNK, _CHUNK)], mem[b], smem[b]
            ).start()

        def wait_in(b):
            pltpu.make_async_copy(
                bits_sh.at[pl.ds(0, crow)], rows[b], sin[b]).wait()
            pltpu.make_async_copy(
                mem_hbm.at[pl.ds(0, _CHUNK)], mem[b], smem[b]).wait()

        def issue_out(ci, b):
            pltpu.make_async_copy(
                out[b], out_hbm.at[pl.ds(n0 + ci * _CHUNK, _CHUNK)], sout[b]
            ).start()

        def wait_out(b):
            pltpu.make_async_copy(
                out[b], out_hbm.at[pl.ds(0, _CHUNK)], sout[b]).wait()

        def compute(b):
            @plsc.parallel_loop(0, _CHUNK * _GROUPS, 1, unroll=2)
            def _group(i):
                q = i // _GROUPS
                t = i % _GROUPS
                qvec = jnp.broadcast_to(q, (_LANES,)).astype(jnp.int32)
                two_iota = jnp.arange(_LANES, dtype=jnp.int32) * 2
                if True:
                    n0a = n1a = n2a = None
                    for k in range(_NBITS):
                        w = rows[b][q * _NBITS + k, pl.ds(t * _LANES, _LANES)]
                        if k < 4:
                            n0a = (w << k) if n0a is None else n0a + (w << k)
                        elif k < 8:
                            n1a = (w << (k - 4)) if n1a is None else n1a + (w << (k - 4))
                        else:
                            n2a = (w << (k - 8)) if n2a is None else n2a + (w << (k - 8))
                    for h in range(2):
                        word = None
                        for s in range(4):
                            j = 4 * h + s
                            p0 = (n0a >> (4 * j)) & 0xF if j else n0a & 0xF
                            if j == 0:
                                p1 = (n1a << 4) & 0xF0
                                p2 = (n2a << 8) & 0xF00
                            elif j == 1:
                                p1 = n1a & 0xF0
                                p2 = (n2a << 4) & 0xF00
                            else:
                                p1 = (n1a >> (4 * j - 4)) & 0xF0
                                p2 = (n2a >> (4 * j - 8)) & 0xF00
                            addr = p0 | p1 | p2
                            vals = plsc.load_gather(mem[b], [qvec, addr])
                            bit = jnp.where(vals > 0.5, 1, 0).astype(jnp.int32)
                            word = bit if s == 0 else word | (bit << (8 * s))
                        plsc.store_scatter(
                            out[b], [qvec, two_iota + (t * 32 + h)], word)

        issue_in(0, 0)

        def body(h, carry):
            c0 = h * 2
            issue_in(c0 + 1, 1)
            wait_in(0)

            @pl.when(h > 0)
            def _():
                wait_out(0)

            compute(0)
            issue_out(c0, 0)

            @pl.when(h < nchunks // 2 - 1)
            def _():
                issue_in(c0 + 2, 0)

            wait_in(1)

            @pl.when(h > 0)
            def _():
                wait_out(1)

            compute(1)
            issue_out(c0 + 1, 1)
            return carry

        lax.fori_loop(0, nchunks // 2, body, 0)
        wait_out(0)
        wait_out(1)

    return ram_kernel


def kernel(input_bits, connections, memory):
    shifts = (jnp.arange(8, dtype=jnp.int32) * 4)[None, :, None]
    bits_packed = (
        (input_bits.reshape(_WORDS, 8, _TOTAL_BITS) << shifts)
        .sum(axis=1, dtype=jnp.int32).T)
    conn_flat = connections.reshape(-1)
    packed = _build_sc_kernel()(bits_packed, conn_flat, memory)
    by = jax.lax.bitcast_convert_type(packed, jnp.int8)
    return by.reshape(_NEURONS, _BATCH).T.astype(jnp.bool_)
